# Initial kernel scaffold; baseline (speedup 1.0000x reference)
import functools

import jax
import jax.numpy as jnp
from jax import lax
from jax.experimental import pallas as pl
from jax.experimental.pallas import tpu as pltpu
from jax.experimental.pallas import tpu_sc as plsc

N = 10000
D = 128
H = 64
E = 320000
L = 16
NC, NS = 2, 16
WIN = 128
E2 = 323584
WPT = E2 // NS // WIN
NPAD = 10240

_R1 = 2000


def _dense_body(x_ref, wfc_ref, wattn_ref, lw_ref, z2_ref, zl_ref, p_ref, q_ref):
    x = x_ref[...]
    z = lax.dot_general(x, wfc_ref[...], (((1,), (1,)), ((), ())),
                        preferred_element_type=jnp.float32)
    zl_ref[...] = lax.dot_general(z, lw_ref[...], (((1,), (0,)), ((), ())),
                                  preferred_element_type=jnp.float32)
    wa = wattn_ref[...]
    p_ref[...] = jnp.sum(z * wa[:, :D], axis=1).reshape(1, 1, _R1)
    q_ref[...] = jnp.sum(z * wa[:, D:2 * D], axis=1).reshape(1, 1, _R1)
    z2_ref[0] = z[:, :H]
    z2_ref[1] = z[:, H:]


def _dense(x, W_fc, W_attn, loop_weight):
    nb = N // _R1
    return pl.pallas_call(
        _dense_body,
        grid=(nb,),
        in_specs=[
            pl.BlockSpec((_R1, D), lambda i: (i, 0)),
            pl.BlockSpec((D, D), lambda i: (0, 0)),
            pl.BlockSpec((1, 3 * D), lambda i: (0, 0)),
            pl.BlockSpec((D, D), lambda i: (0, 0)),
        ],
        out_specs=[
            pl.BlockSpec((NC, _R1, H), lambda i: (0, i, 0)),
            pl.BlockSpec((_R1, D), lambda i: (i, 0)),
            pl.BlockSpec((1, 1, _R1), lambda i: (i, 0, 0)),
            pl.BlockSpec((1, 1, _R1), lambda i: (i, 0, 0)),
        ],
        out_shape=[
            jax.ShapeDtypeStruct((NC, N, H), jnp.float32),
            jax.ShapeDtypeStruct((N, D), jnp.float32),
            jax.ShapeDtypeStruct((nb, 1, _R1), jnp.float32),
            jax.ShapeDtypeStruct((nb, 1, _R1), jnp.float32),
        ],
    )(x, W_fc, W_attn, loop_weight)


_R2 = 3200


def _rproj_body(rh_ref, wattn_ref, r_ref):
    w3 = wattn_ref[:, 2 * D:]
    r_ref[...] = jnp.sum(rh_ref[...] * w3, axis=1).reshape(1, 1, _R2)


def _rproj(r_h, W_attn):
    nb = E // _R2
    return pl.pallas_call(
        _rproj_body,
        grid=(nb,),
        in_specs=[
            pl.BlockSpec((_R2, D), lambda i: (i, 0)),
            pl.BlockSpec((1, 3 * D), lambda i: (0, 0)),
        ],
        out_specs=pl.BlockSpec((1, 1, _R2), lambda i: (i, 0, 0)),
        out_shape=jax.ShapeDtypeStruct((nb, 1, _R2), jnp.float32),
    )(r_h, W_attn)


def _splat(v, k):
    idx = jnp.full((L,), k, dtype=jnp.int32)
    return jnp.take(v, idx, axis=0, mode="promise_in_bounds")


def _sc_edge_body(src_h, dst_h, r_h, p_h, q_h, z2_h, agg_h,
                  z_sp, agg_sp, s_sp,
                  p_v, q_v, idxs, idxd, rbuf, exb, rows, zb, sv_buf, abuf, sem):
    c = lax.axis_index("c")
    t = lax.axis_index("s")

    pltpu.sync_copy(p_h, p_v)
    pltpu.sync_copy(q_h, q_v)

    @pl.when(t == 0)
    def _():
        pltpu.sync_copy(z2_h.at[c], z_sp)

    def _zb(i, _):
        for cg in range(H // L):
            zb[i, pl.ds(cg * L, L)] = jnp.zeros((L,), jnp.float32)
        return 0
    lax.fori_loop(0, WIN, _zb, 0)
    for i in range(5):
        pltpu.sync_copy(zb, agg_sp.at[pl.ds(t * 640 + i * WIN, WIN)])
    for i in range(10):
        pltpu.sync_copy(zb.at[0], s_sp.at[pl.ds(t * 64 + NS * 64 * i, 64)])

    plsc.subcore_barrier()

    def _window(j, _):
        off = t * (WPT * WIN) + j * WIN
        pltpu.sync_copy(src_h.at[pl.ds(off, WIN)], idxs.at[0])
        pltpu.sync_copy(dst_h.at[pl.ds(off, WIN)], idxd.at[0])
        pltpu.sync_copy(r_h.at[pl.ds(off, WIN)], rbuf)
        gat = pltpu.make_async_copy(z_sp.at[idxs.at[0]], rows, sem)
        gat.start()
        for g in range(WIN // L):
            sl = pl.ds(g * L, L)
            iv_s = idxs[0, sl]
            iv_d = idxd[0, sl]
            a = (plsc.load_gather(p_v, [iv_s]) + plsc.load_gather(q_v, [iv_d])
                 + rbuf[sl])
            exb[sl] = jnp.exp(jnp.maximum(a, a * 0.01))
        pltpu.sync_copy(exb, s_sp.at[idxd.at[0]], add=True)
        gat.wait()
        for g in range(WIN // L):
            exv = exb[pl.ds(g * L, L)]
            for k in range(L):
                ei = g * L + k
                scale = _splat(exv, k)
                for cg in range(H // L):
                    sl = pl.ds(cg * L, L)
                    rows[ei, sl] = rows[ei, sl] * scale
        pltpu.sync_copy(rows, agg_sp.at[idxd.at[0]], add=True)
        return 0

    lax.fori_loop(0, WPT, _window, 0)
    plsc.subcore_barrier()

    row0 = t * 640
    pltpu.sync_copy(s_sp.at[pl.ds(row0, 640)], sv_buf)
    pltpu.sync_copy(agg_sp.at[pl.ds(row0, 640)], abuf)

    def _norm(g, _):
        sv = sv_buf[pl.ds(g * L, L)]
        inv = jnp.where(sv > 0.0, 1.0 / sv, 0.0)
        for k in range(L):
            scale = _splat(inv, k)
            for cg in range(H // L):
                sl = pl.ds(cg * L, L)
                abuf[g * L + k, sl] = abuf[g * L + k, sl] * scale
        return 0

    lax.fori_loop(0, 640 // L, _norm, 0)
    pltpu.sync_copy(abuf, agg_h.at[c, pl.ds(row0, 640)])


def _sc_edge(srcp, dstp, rp, p, q, z2):
    mesh = plsc.VectorSubcoreMesh(core_axis_name="c", subcore_axis_name="s",
                                  num_cores=NC, num_subcores=NS)
    f = functools.partial(
        pl.kernel,
        out_type=jax.ShapeDtypeStruct((NC, NPAD, H), jnp.float32),
        mesh=mesh,
        scratch_types=[
            pltpu.VMEM_SHARED((N, H), jnp.float32),
            pltpu.VMEM_SHARED((NPAD, H), jnp.float32),
            pltpu.VMEM_SHARED((NPAD,), jnp.float32),
            pltpu.VMEM((N,), jnp.float32),
            pltpu.VMEM((N,), jnp.float32),
            pltpu.VMEM((1, WIN), jnp.int32),
            pltpu.VMEM((1, WIN), jnp.int32),
            pltpu.VMEM((WIN,), jnp.float32),
            pltpu.VMEM((WIN,), jnp.float32),
            pltpu.VMEM((WIN, H), jnp.float32),
            pltpu.VMEM((WIN, H), jnp.float32),
            pltpu.VMEM((640,), jnp.float32),
            pltpu.VMEM((640, H), jnp.float32),
            pltpu.SemaphoreType.DMA,
        ],
    )(_sc_edge_body)
    return f(srcp, dstp, rp, p, q, z2)


def _final_body(agg_ref, zl_ref, h_ref):
    a = jnp.concatenate([agg_ref[0], agg_ref[1]], axis=1)
    h_ref[...] = jnp.maximum(a + zl_ref[...], 0.0)


def _final(agg, zl):
    nb = N // _R1
    return pl.pallas_call(
        _final_body,
        grid=(nb,),
        in_specs=[
            pl.BlockSpec((NC, _R1, H), lambda i: (0, i, 0)),
            pl.BlockSpec((_R1, D), lambda i: (i, 0)),
        ],
        out_specs=pl.BlockSpec((_R1, D), lambda i: (i, 0)),
        out_shape=jax.ShapeDtypeStruct((N, D), jnp.float32),
    )(agg, zl)


def kernel(x, edge_index, r_h, W_fc, W_attn, loop_weight):
    z2, zl, p3, q3 = _dense(x, W_fc, W_attn, loop_weight)
    r3 = _rproj(r_h, W_attn)
    p = p3.reshape(N)
    q = q3.reshape(N)
    ei = edge_index.astype(jnp.int32)
    pad = E2 - E
    srcp = jnp.concatenate([ei[0], jnp.zeros((pad,), jnp.int32)])
    dstp = jnp.concatenate([ei[1], jnp.zeros((pad,), jnp.int32)])
    rp = jnp.concatenate([r3.reshape(E), jnp.full((pad,), -1e30, jnp.float32)])
    agg = _sc_edge(srcp, dstp, rp, p, q, z2)
    return _final(agg, zl)


# speedup vs baseline: 10.4266x; 10.4266x over previous
import functools

import jax
import jax.numpy as jnp
from jax import lax
from jax.experimental import pallas as pl
from jax.experimental.pallas import tpu as pltpu
from jax.experimental.pallas import tpu_sc as plsc

N = 10000
D = 128
E = 320000
L = 16
NC, NS = 2, 16
WIN = 128
E2 = 323584
WPT = E2 // (NC * NS * WIN)
NPAD = 10240

_R1 = 2000


def _dense_body(x_ref, wfc_ref, wattn_ref, lw_ref, z_ref, zl_ref, p_ref, q_ref):
    x = x_ref[...]
    z = lax.dot_general(x, wfc_ref[...], (((1,), (1,)), ((), ())),
                        preferred_element_type=jnp.float32)
    zl_ref[...] = lax.dot_general(z, lw_ref[...], (((1,), (0,)), ((), ())),
                                  preferred_element_type=jnp.float32)
    wa = wattn_ref[...]
    p_ref[...] = jnp.sum(z * wa[:, :D], axis=1).reshape(1, 1, _R1)
    q_ref[...] = jnp.sum(z * wa[:, D:2 * D], axis=1).reshape(1, 1, _R1)
    z_ref[...] = z


def _dense(x, W_fc, W_attn, loop_weight):
    nb = N // _R1
    return pl.pallas_call(
        _dense_body,
        grid=(nb,),
        in_specs=[
            pl.BlockSpec((_R1, D), lambda i: (i, 0)),
            pl.BlockSpec((D, D), lambda i: (0, 0)),
            pl.BlockSpec((1, 3 * D), lambda i: (0, 0)),
            pl.BlockSpec((D, D), lambda i: (0, 0)),
        ],
        out_specs=[
            pl.BlockSpec((_R1, D), lambda i: (i, 0)),
            pl.BlockSpec((_R1, D), lambda i: (i, 0)),
            pl.BlockSpec((1, 1, _R1), lambda i: (i, 0, 0)),
            pl.BlockSpec((1, 1, _R1), lambda i: (i, 0, 0)),
        ],
        out_shape=[
            jax.ShapeDtypeStruct((N, D), jnp.float32),
            jax.ShapeDtypeStruct((N, D), jnp.float32),
            jax.ShapeDtypeStruct((nb, 1, _R1), jnp.float32),
            jax.ShapeDtypeStruct((nb, 1, _R1), jnp.float32),
        ],
    )(x, W_fc, W_attn, loop_weight)


_R2 = 3200


def _rproj_body(rh_ref, wattn_ref, r_ref):
    w3 = wattn_ref[:, 2 * D:]
    r_ref[...] = jnp.sum(rh_ref[...] * w3, axis=1).reshape(1, 1, _R2)


def _rproj(r_h, W_attn):
    nb = E // _R2
    return pl.pallas_call(
        _rproj_body,
        grid=(nb,),
        in_specs=[
            pl.BlockSpec((_R2, D), lambda i: (i, 0)),
            pl.BlockSpec((1, 3 * D), lambda i: (0, 0)),
        ],
        out_specs=pl.BlockSpec((1, 1, _R2), lambda i: (i, 0, 0)),
        out_shape=jax.ShapeDtypeStruct((nb, 1, _R2), jnp.float32),
    )(r_h, W_attn)


def _splat(v, k):
    idx = jnp.full((L, 1), k, dtype=jnp.int32)
    dn = lax.GatherDimensionNumbers(offset_dims=(), collapsed_slice_dims=(0,),
                                    start_index_map=(0,))
    return lax.gather(v, idx, dn, (1,),
                      mode=lax.GatherScatterMode.PROMISE_IN_BOUNDS)


def _sc_edge_body(src_h, dst_h, r_h, p_h, q_h, z_h, agg_h, s_h,
                  agg_sp, s_sp,
                  p_v, q_v, idxs, idxd, rbuf, exb, rows, sem):
    c = lax.axis_index("c")
    t = lax.axis_index("s")

    pltpu.sync_copy(p_h, p_v)
    pltpu.sync_copy(q_h, q_v)

    def _zb(i, _):
        for cg in range(D // L):
            rows[i, pl.ds(cg * L, L)] = jnp.zeros((L,), jnp.float32)
        return 0
    lax.fori_loop(0, WIN, _zb, 0)
    for i in range(5):
        pltpu.sync_copy(rows, agg_sp.at[pl.ds(t * 640 + i * WIN, WIN)])
        pltpu.sync_copy(rows.at[0], s_sp.at[pl.ds(t * 640 + i * WIN, WIN)])

    plsc.subcore_barrier()

    base = (c * NS + t) * (WPT * WIN)

    def _window(j, _):
        off = base + j * WIN
        pltpu.sync_copy(src_h.at[pl.ds(off, WIN)], idxs)
        pltpu.sync_copy(dst_h.at[pl.ds(off, WIN)], idxd)
        pltpu.sync_copy(r_h.at[pl.ds(off, WIN)], rbuf)
        gat = pltpu.make_async_copy(z_h.at[idxs], rows, sem)
        gat.start()
        for g in range(WIN // L):
            sl = pl.ds(g * L, L)
            iv_s = idxs[sl]
            iv_d = idxd[sl]
            a = (plsc.load_gather(p_v, [iv_s]) + plsc.load_gather(q_v, [iv_d])
                 + rbuf[sl])
            exb[sl] = jnp.exp(jnp.maximum(a, a * 0.01))
        pltpu.sync_copy(exb, s_sp.at[idxd], add=True)
        gat.wait()
        for g in range(WIN // L):
            exv = exb[pl.ds(g * L, L)]
            for k in range(L):
                ei = g * L + k
                scale = _splat(exv, k)
                for cg in range(D // L):
                    sl = pl.ds(cg * L, L)
                    rows[ei, sl] = rows[ei, sl] * scale
        pltpu.sync_copy(rows, agg_sp.at[idxd], add=True)
        return 0

    lax.fori_loop(0, WPT, _window, 0)
    plsc.subcore_barrier()

    row0 = t * 640
    pltpu.sync_copy(agg_sp.at[pl.ds(row0, 640)],
                    agg_h.at[pl.ds(c * NPAD + row0, 640)])
    pltpu.sync_copy(s_sp.at[pl.ds(row0, 640)],
                    s_h.at[pl.ds(c * NPAD + row0, 640)])


def _sc_edge(srcp, dstp, rp, p, q, z):
    mesh = plsc.VectorSubcoreMesh(core_axis_name="c", subcore_axis_name="s",
                                  num_cores=NC, num_subcores=NS)
    f = functools.partial(
        pl.kernel,
        out_type=[jax.ShapeDtypeStruct((NC * NPAD, D), jnp.float32),
                  jax.ShapeDtypeStruct((NC * NPAD,), jnp.float32)],
        mesh=mesh,
        compiler_params=pltpu.CompilerParams(needs_layout_passes=False),
        scratch_types=[
            pltpu.VMEM_SHARED((NPAD, D), jnp.float32),
            pltpu.VMEM_SHARED((NPAD,), jnp.float32),
            pltpu.VMEM((N,), jnp.float32),
            pltpu.VMEM((N,), jnp.float32),
            pltpu.VMEM((WIN,), jnp.int32),
            pltpu.VMEM((WIN,), jnp.int32),
            pltpu.VMEM((WIN,), jnp.float32),
            pltpu.VMEM((WIN,), jnp.float32),
            pltpu.VMEM((WIN, D), jnp.float32),
            pltpu.SemaphoreType.DMA,
        ],
    )(_sc_edge_body)
    return f(srcp, dstp, rp, p, q, z)


_RF = 80


def _final_body(a0_ref, a1_ref, s0_ref, s1_ref, zl_ref, h_ref):
    s = s0_ref[...] + s1_ref[...]
    inv = jnp.where(s > 0.0, 1.0 / s, 0.0)[0]
    r_i = lax.broadcasted_iota(jnp.int32, (_RF, _RF), 0)
    c_i = lax.broadcasted_iota(jnp.int32, (_RF, _RF), 1)
    diagm = jnp.where(r_i == c_i, inv, 0.0)
    a = a0_ref[...] + a1_ref[...]
    scaled = lax.dot_general(diagm, a, (((1,), (0,)), ((), ())),
                             preferred_element_type=jnp.float32)
    h_ref[...] = jnp.maximum(scaled + zl_ref[...], 0.0)


def _final(agg, s, zl):
    nb = N // _RF
    off = NPAD // _RF
    s3 = s.reshape(NC * NPAD // _RF, 1, _RF)
    return pl.pallas_call(
        _final_body,
        grid=(nb,),
        in_specs=[
            pl.BlockSpec((_RF, D), lambda i: (i, 0)),
            pl.BlockSpec((_RF, D), lambda i: (i + off, 0)),
            pl.BlockSpec((1, 1, _RF), lambda i: (i, 0, 0)),
            pl.BlockSpec((1, 1, _RF), lambda i: (i + off, 0, 0)),
            pl.BlockSpec((_RF, D), lambda i: (i, 0)),
        ],
        out_specs=pl.BlockSpec((_RF, D), lambda i: (i, 0)),
        out_shape=jax.ShapeDtypeStruct((N, D), jnp.float32),
    )(agg, agg, s3, s3, zl)


def kernel(x, edge_index, r_h, W_fc, W_attn, loop_weight):
    z, zl, p3, q3 = _dense(x, W_fc, W_attn, loop_weight)
    r3 = _rproj(r_h, W_attn)
    p = p3.reshape(N)
    q = q3.reshape(N)
    ei = edge_index.astype(jnp.int32)
    pad = E2 - E
    srcp = jnp.concatenate([ei[0], jnp.zeros((pad,), jnp.int32)])
    dstp = jnp.concatenate([ei[1], jnp.zeros((pad,), jnp.int32)])
    rp = jnp.concatenate([r3.reshape(E), jnp.full((pad,), -1e30, jnp.float32)])
    agg, s = _sc_edge(srcp, dstp, rp, p, q, z)
    return _final(agg, s, zl)


# speedup vs baseline: 12.0034x; 1.1512x over previous
import functools

import jax
import jax.numpy as jnp
from jax import lax
from jax.experimental import pallas as pl
from jax.experimental.pallas import tpu as pltpu
from jax.experimental.pallas import tpu_sc as plsc

N = 10000
D = 128
E = 320000
L = 16
NC, NS = 2, 16
WIN = 128
E2 = 323584
WPT = E2 // (NC * NS * WIN)
NPAD = 10240

_R1 = 2000


def _dense_body(x_ref, wfc_ref, wattn_ref, lw_ref, z_ref, zl_ref, p_ref, q_ref):
    x = x_ref[...]
    z = lax.dot_general(x, wfc_ref[...], (((1,), (1,)), ((), ())),
                        preferred_element_type=jnp.float32)
    zl_ref[...] = lax.dot_general(z, lw_ref[...], (((1,), (0,)), ((), ())),
                                  preferred_element_type=jnp.float32)
    wa = wattn_ref[...]
    p_ref[...] = jnp.sum(z * wa[:, :D], axis=1).reshape(1, 1, _R1)
    q_ref[...] = jnp.sum(z * wa[:, D:2 * D], axis=1).reshape(1, 1, _R1)
    z_ref[...] = z


def _dense(x, W_fc, W_attn, loop_weight):
    nb = N // _R1
    return pl.pallas_call(
        _dense_body,
        grid=(nb,),
        in_specs=[
            pl.BlockSpec((_R1, D), lambda i: (i, 0)),
            pl.BlockSpec((D, D), lambda i: (0, 0)),
            pl.BlockSpec((1, 3 * D), lambda i: (0, 0)),
            pl.BlockSpec((D, D), lambda i: (0, 0)),
        ],
        out_specs=[
            pl.BlockSpec((_R1, D), lambda i: (i, 0)),
            pl.BlockSpec((_R1, D), lambda i: (i, 0)),
            pl.BlockSpec((1, 1, _R1), lambda i: (i, 0, 0)),
            pl.BlockSpec((1, 1, _R1), lambda i: (i, 0, 0)),
        ],
        out_shape=[
            jax.ShapeDtypeStruct((N, D), jnp.float32),
            jax.ShapeDtypeStruct((N, D), jnp.float32),
            jax.ShapeDtypeStruct((nb, 1, _R1), jnp.float32),
            jax.ShapeDtypeStruct((nb, 1, _R1), jnp.float32),
        ],
    )(x, W_fc, W_attn, loop_weight)


_R2 = 3200


def _rproj_body(rh_ref, wattn_ref, r_ref):
    w3 = wattn_ref[:, 2 * D:]
    r_ref[...] = jnp.sum(rh_ref[...] * w3, axis=1).reshape(1, 1, _R2)


def _rproj(r_h, W_attn):
    nb = E // _R2
    return pl.pallas_call(
        _rproj_body,
        grid=(nb,),
        in_specs=[
            pl.BlockSpec((_R2, D), lambda i: (i, 0)),
            pl.BlockSpec((1, 3 * D), lambda i: (0, 0)),
        ],
        out_specs=pl.BlockSpec((1, 1, _R2), lambda i: (i, 0, 0)),
        out_shape=jax.ShapeDtypeStruct((nb, 1, _R2), jnp.float32),
    )(r_h, W_attn)


def _splat(v, k):
    idx = jnp.full((L, 1), k, dtype=jnp.int32)
    dn = lax.GatherDimensionNumbers(offset_dims=(), collapsed_slice_dims=(0,),
                                    start_index_map=(0,))
    return lax.gather(v, idx, dn, (1,),
                      mode=lax.GatherScatterMode.PROMISE_IN_BOUNDS)


def _sc_edge_body(src_h, dst_h, r_h, p_h, q_h, z_h, agg_h, s_h,
                  agg_sp, s_sp,
                  p_v, q_v, idxs_a, idxd_a, rbuf_a, idxs_b, idxd_b, rbuf_b,
                  exb, rows, semg, sa1, sa2, sa3, sb1, sb2, sb3):
    c = lax.axis_index("c")
    t = lax.axis_index("s")

    pltpu.sync_copy(p_h, p_v)
    pltpu.sync_copy(q_h, q_v)

    def _zb(i, _):
        for cg in range(D // L):
            rows[i, pl.ds(cg * L, L)] = jnp.zeros((L,), jnp.float32)
        return 0
    lax.fori_loop(0, WIN, _zb, 0)
    for i in range(5):
        pltpu.sync_copy(rows, agg_sp.at[pl.ds(t * 640 + i * WIN, WIN)])
        pltpu.sync_copy(rows.at[0], s_sp.at[pl.ds(t * 640 + i * WIN, WIN)])

    plsc.subcore_barrier()

    base = (c * NS + t) * (WPT * WIN)

    def _loads(off, bi, bd, br, s1, s2, s3, start):
        d1 = pltpu.make_async_copy(src_h.at[pl.ds(off, WIN)], bi, s1)
        d2 = pltpu.make_async_copy(dst_h.at[pl.ds(off, WIN)], bd, s2)
        d3 = pltpu.make_async_copy(r_h.at[pl.ds(off, WIN)], br, s3)
        for d in (d1, d2, d3):
            if start:
                d.start()
            else:
                d.wait()

    def _phase(off, bi, bd, br, s1, s2, s3, pref):
        _loads(off, bi, bd, br, s1, s2, s3, False)
        gat = pltpu.make_async_copy(z_h.at[bi], rows, semg)
        gat.start()
        if pref is not None:
            _loads(*pref, True)

        def _exp(g, _):
            sl = pl.ds(g * L, L)
            a = (plsc.load_gather(p_v, [bi[sl]]) + plsc.load_gather(q_v, [bd[sl]])
                 + br[sl])
            exb[sl] = jnp.exp(jnp.maximum(a, a * 0.01))
            return 0
        lax.fori_loop(0, WIN // L, _exp, 0)
        pltpu.sync_copy(exb, s_sp.at[bd], add=True)
        gat.wait()

        def _scale(g, _):
            exv = exb[pl.ds(g * L, L)]
            for k in range(L):
                scale = _splat(exv, k)
                for cg in range(D // L):
                    sl = pl.ds(cg * L, L)
                    rows[g * L + k, sl] = rows[g * L + k, sl] * scale
            return 0
        lax.fori_loop(0, WIN // L, _scale, 0)
        pltpu.sync_copy(rows, agg_sp.at[bd], add=True)

    bufa = (idxs_a, idxd_a, rbuf_a, sa1, sa2, sa3)
    bufb = (idxs_b, idxd_b, rbuf_b, sb1, sb2, sb3)
    _loads(base, *bufa, True)

    def _pair(m, _):
        ja = base + (2 * m) * WIN
        jb = ja + WIN
        _phase(ja, *bufa, pref=(jb,) + bufb)
        _phase(jb, *bufb, pref=(ja + 2 * WIN,) + bufa)
        return 0

    lax.fori_loop(0, (WPT - 1) // 2, _pair, 0)
    _phase(base + (WPT - 1) * WIN, *bufa, pref=None)
    plsc.subcore_barrier()

    row0 = t * 640
    pltpu.sync_copy(agg_sp.at[pl.ds(row0, 640)],
                    agg_h.at[pl.ds(c * NPAD + row0, 640)])
    pltpu.sync_copy(s_sp.at[pl.ds(row0, 640)],
                    s_h.at[pl.ds(c * NPAD + row0, 640)])


def _sc_edge(srcp, dstp, rp, p, q, z):
    mesh = plsc.VectorSubcoreMesh(core_axis_name="c", subcore_axis_name="s",
                                  num_cores=NC, num_subcores=NS)
    f = functools.partial(
        pl.kernel,
        out_type=[jax.ShapeDtypeStruct((NC * NPAD, D), jnp.float32),
                  jax.ShapeDtypeStruct((NC * NPAD,), jnp.float32)],
        mesh=mesh,
        compiler_params=pltpu.CompilerParams(needs_layout_passes=False),
        scratch_types=[
            pltpu.VMEM_SHARED((NPAD, D), jnp.float32),
            pltpu.VMEM_SHARED((NPAD,), jnp.float32),
            pltpu.VMEM((N,), jnp.float32),
            pltpu.VMEM((N,), jnp.float32),
            pltpu.VMEM((WIN,), jnp.int32),
            pltpu.VMEM((WIN,), jnp.int32),
            pltpu.VMEM((WIN,), jnp.float32),
            pltpu.VMEM((WIN,), jnp.int32),
            pltpu.VMEM((WIN,), jnp.int32),
            pltpu.VMEM((WIN,), jnp.float32),
            pltpu.VMEM((WIN,), jnp.float32),
            pltpu.VMEM((WIN, D), jnp.float32),
            pltpu.SemaphoreType.DMA,
            pltpu.SemaphoreType.DMA,
            pltpu.SemaphoreType.DMA,
            pltpu.SemaphoreType.DMA,
            pltpu.SemaphoreType.DMA,
            pltpu.SemaphoreType.DMA,
            pltpu.SemaphoreType.DMA,
        ],
    )(_sc_edge_body)
    return f(srcp, dstp, rp, p, q, z)


_RF = 80


def _final_body(a0_ref, a1_ref, s0_ref, s1_ref, zl_ref, h_ref):
    s = s0_ref[...] + s1_ref[...]
    inv = jnp.where(s > 0.0, 1.0 / s, 0.0)[0]
    r_i = lax.broadcasted_iota(jnp.int32, (_RF, _RF), 0)
    c_i = lax.broadcasted_iota(jnp.int32, (_RF, _RF), 1)
    diagm = jnp.where(r_i == c_i, inv, 0.0)
    a = a0_ref[...] + a1_ref[...]
    scaled = lax.dot_general(diagm, a, (((1,), (0,)), ((), ())),
                             preferred_element_type=jnp.float32)
    h_ref[...] = jnp.maximum(scaled + zl_ref[...], 0.0)


def _final(agg, s, zl):
    nb = N // _RF
    off = NPAD // _RF
    s3 = s.reshape(NC * NPAD // _RF, 1, _RF)
    return pl.pallas_call(
        _final_body,
        grid=(nb,),
        in_specs=[
            pl.BlockSpec((_RF, D), lambda i: (i, 0)),
            pl.BlockSpec((_RF, D), lambda i: (i + off, 0)),
            pl.BlockSpec((1, 1, _RF), lambda i: (i, 0, 0)),
            pl.BlockSpec((1, 1, _RF), lambda i: (i + off, 0, 0)),
            pl.BlockSpec((_RF, D), lambda i: (i, 0)),
        ],
        out_specs=pl.BlockSpec((_RF, D), lambda i: (i, 0)),
        out_shape=jax.ShapeDtypeStruct((N, D), jnp.float32),
    )(agg, agg, s3, s3, zl)


def kernel(x, edge_index, r_h, W_fc, W_attn, loop_weight):
    z, zl, p3, q3 = _dense(x, W_fc, W_attn, loop_weight)
    r3 = _rproj(r_h, W_attn)
    p = p3.reshape(N)
    q = q3.reshape(N)
    ei = edge_index.astype(jnp.int32)
    pad = E2 - E
    srcp = jnp.concatenate([ei[0], jnp.zeros((pad,), jnp.int32)])
    dstp = jnp.concatenate([ei[1], jnp.zeros((pad,), jnp.int32)])
    rp = jnp.concatenate([r3.reshape(E), jnp.full((pad,), -1e30, jnp.float32)])
    agg, s = _sc_edge(srcp, dstp, rp, p, q, z)
    return _final(agg, s, zl)
